# baseline (speedup 1.0000x reference)
import functools

import jax
import jax.numpy as jnp
from jax import lax
from jax.experimental import pallas as pl
from jax.experimental.pallas import tpu as pltpu
from jax.experimental.pallas import tpu_sc as plsc

N_ATOMS = 262144
D_OUT = 366
D_EMB = 87
D_PAD = 368
CHUNK = 64
LANES = 16


@functools.lru_cache(maxsize=1)
def _build():
    info = plsc.get_sparse_core_info()
    nw = info.num_cores * info.num_subcores
    rows_per_w = N_ATOMS // nw
    n_chunks = rows_per_w // CHUNK
    groups = CHUNK // LANES
    cwords = CHUNK * D_OUT
    cbuf_words = cwords + 8

    mesh = plsc.VectorSubcoreMesh(core_axis_name="c", subcore_axis_name="s")

    @functools.partial(
        pl.kernel,
        mesh=mesh,
        compiler_params=pltpu.CompilerParams(
            use_tc_tiling_on_sc=False, needs_layout_passes=False
        ),
        out_type=jax.ShapeDtypeStruct((N_ATOMS * D_OUT,), jnp.float32),
        scratch_types=[
            pltpu.VMEM((D_EMB * D_PAD,), jnp.float32),
            pltpu.VMEM((rows_per_w,), jnp.int32),
            pltpu.VMEM((cbuf_words,), jnp.float32),
        ],
    )
    def k(z_hbm, table_hbm, out_hbm, table_v, zslab, cbuf):
        wid = lax.axis_index("s") * info.num_cores + lax.axis_index("c")
        row0 = wid * rows_per_w

        pltpu.sync_copy(table_hbm, table_v)
        pltpu.sync_copy(z_hbm.at[pl.ds(row0, rows_per_w)], zslab)

        lane_rows = lax.iota(jnp.int32, LANES) * D_OUT

        def chunk_body(i, carry):
            for g in range(groups):
                rg = (i * groups + g) * LANES
                zv = zslab[pl.ds(rg, LANES)]
                srcb = zv * D_PAD
                dstb = lane_rows + (g * LANES * D_OUT)

                def col_body(t, c2):
                    c = (D_PAD - 1) - t
                    x = plsc.load_gather(table_v, [srcb + c])
                    plsc.store_scatter(cbuf, [dstb + c], x)
                    return c2

                lax.fori_loop(0, D_PAD, col_body, 0, unroll=8)
            pltpu.sync_copy(
                cbuf.at[pl.ds(0, cwords)],
                out_hbm.at[pl.ds((row0 + i * CHUNK) * D_OUT, cwords)],
            )
            return carry

        lax.fori_loop(0, n_chunks, chunk_body, 0)

    return k


def kernel(Z, emb_table):
    z_flat = Z.reshape(-1)
    table = jnp.zeros((D_EMB, D_PAD), jnp.float32).at[:, :D_EMB].set(emb_table)
    out = _build()(z_flat, table.reshape(-1))
    return out.reshape(N_ATOMS, D_OUT)


# speedup vs baseline: 2.0606x; 2.0606x over previous
import functools

import jax
import jax.numpy as jnp
from jax import lax
from jax.experimental import pallas as pl
from jax.experimental.pallas import tpu as pltpu
from jax.experimental.pallas import tpu_sc as plsc

N_ATOMS = 262144
D_OUT = 366
D_EMB = 87
D_PAD = 368
CHUNK = 64
LANES = 16


@functools.lru_cache(maxsize=1)
def _build():
    info = plsc.get_sparse_core_info()
    nw = info.num_cores * info.num_subcores
    rows_per_w = N_ATOMS // nw
    n_chunks = rows_per_w // CHUNK
    n_pairs = n_chunks // 2
    groups = CHUNK // LANES
    cwords = CHUNK * D_OUT

    mesh = plsc.VectorSubcoreMesh(core_axis_name="c", subcore_axis_name="s")

    @functools.partial(
        pl.kernel,
        mesh=mesh,
        compiler_params=pltpu.CompilerParams(
            use_tc_tiling_on_sc=False, needs_layout_passes=False
        ),
        out_type=jax.ShapeDtypeStruct((N_ATOMS * D_OUT,), jnp.float32),
        scratch_types=[
            pltpu.VMEM((D_EMB * D_PAD,), jnp.float32),
            pltpu.VMEM((rows_per_w,), jnp.int32),
            pltpu.VMEM((cwords,), jnp.float32),
            pltpu.VMEM((cwords,), jnp.float32),
            pltpu.SemaphoreType.DMA,
            pltpu.SemaphoreType.DMA,
        ],
    )
    def k(z_hbm, table_hbm, out_hbm, table_v, zslab, cbuf0, cbuf1, sem0, sem1):
        wid = lax.axis_index("s") * info.num_cores + lax.axis_index("c")
        row0 = wid * rows_per_w

        pltpu.sync_copy(table_hbm, table_v)
        pltpu.sync_copy(z_hbm.at[pl.ds(row0, rows_per_w)], zslab)

        zero16 = jnp.zeros((LANES,), jnp.float32)

        def zero_body(j, _):
            cbuf0[pl.ds(j * LANES, LANES)] = zero16
            cbuf1[pl.ds(j * LANES, LANES)] = zero16
            return _

        lax.fori_loop(0, cwords // LANES, zero_body, 0, unroll=8)

        lane_rows = lax.iota(jnp.int32, LANES) * D_OUT

        def fill(buf, i):
            for g in range(groups):
                rg = (i * groups + g) * LANES
                srcb = zslab[pl.ds(rg, LANES)] * D_PAD
                dstb = lane_rows + (g * LANES * D_OUT)
                for c in range(D_EMB):
                    x = plsc.load_gather(table_v, [srcb + c])
                    plsc.store_scatter(buf, [dstb + c], x)

        def start(buf, i, sem):
            return pltpu.async_copy(
                buf, out_hbm.at[pl.ds((row0 + i * CHUNK) * D_OUT, cwords)], sem
            )

        def drain(buf, sem):
            pltpu.make_async_copy(
                buf, out_hbm.at[pl.ds(row0 * D_OUT, cwords)], sem
            ).wait()

        def body(j, _):
            @pl.when(j > 0)
            def _w0():
                drain(cbuf0, sem0)

            fill(cbuf0, 2 * j)
            start(cbuf0, 2 * j, sem0)

            @pl.when(j > 0)
            def _w1():
                drain(cbuf1, sem1)

            fill(cbuf1, 2 * j + 1)
            start(cbuf1, 2 * j + 1, sem1)
            return _

        lax.fori_loop(0, n_pairs, body, 0)
        drain(cbuf0, sem0)
        drain(cbuf1, sem1)

    return k


def kernel(Z, emb_table):
    z_flat = Z.reshape(-1)
    table = jnp.zeros((D_EMB, D_PAD), jnp.float32).at[:, :D_EMB].set(emb_table)
    out = _build()(z_flat, table.reshape(-1))
    return out.reshape(N_ATOMS, D_OUT)
